# baseline (speedup 1.0000x reference)
import functools

import jax
import jax.numpy as jnp
from jax import lax
from jax.experimental import pallas as pl
from jax.experimental.pallas import tpu as pltpu
from jax.experimental.pallas import tpu_sc as plsc

B, N, C, T, K = 8, 1024, 512, 8192, 20
KPAD = 32
NEG = -3.0e38


def _topk_body(g_ref, t_ref, idx_ref):
    g = g_ref[...]
    tbl = t_ref[...]
    logits = lax.dot_general(
        g, tbl, (((1,), (1,)), ((), ())), preferred_element_type=jnp.float32
    )
    col = lax.broadcasted_iota(jnp.int32, (B, T), 1)
    lane = lax.broadcasted_iota(jnp.int32, (B, KPAD), 1)
    acc = jnp.zeros((B, KPAD), jnp.int32)
    for k in range(K):
        m = jnp.max(logits, axis=1, keepdims=True)
        idx = jnp.min(jnp.where(logits == m, col, T), axis=1)
        acc = jnp.where(lane == k, idx[:, None], acc)
        logits = jnp.where(col == idx[:, None], NEG, logits)
    idx_ref[...] = acc


def _topk(g2, table):
    return pl.pallas_call(
        _topk_body,
        out_shape=jax.ShapeDtypeStruct((B, KPAD), jnp.int32),
    )(g2, table)


_NC, _NS = 2, 16
_NW = _NC * _NS
_BPW = (B * KPAD) // _NW


def _gather_body(t_hbm, idx_hbm, out_hbm, idx_v, rows_v, sem):
    wid = lax.axis_index("s") * _NC + lax.axis_index("c")
    base = wid * _BPW
    pltpu.sync_copy(idx_hbm.at[pl.ds(base, _BPW)], idx_v)
    pltpu.async_copy(t_hbm.at[idx_v], rows_v, sem).wait()
    pltpu.sync_copy(rows_v, out_hbm.at[pl.ds(base, _BPW)])


def _sc_gather(table, idx_flat):
    mesh = plsc.VectorSubcoreMesh(core_axis_name="c", subcore_axis_name="s")
    return pl.kernel(
        _gather_body,
        out_type=jax.ShapeDtypeStruct((B * KPAD, C), jnp.float32),
        mesh=mesh,
        scratch_types=[
            pltpu.VMEM((_BPW,), jnp.int32),
            pltpu.VMEM((_BPW, C), jnp.float32),
            pltpu.SemaphoreType.DMA,
        ],
    )(table, idx_flat)


def _agg_body(x_ref, a_ref, o_ref):
    x = x_ref[0]
    a = a_ref[0]
    seg = lax.dot_general(
        x, a, (((1,), (1,)), ((), ())), preferred_element_type=jnp.float32
    )
    lane = lax.broadcasted_iota(jnp.int32, (N, KPAD), 1)
    seg = jnp.where(lane < K, seg, NEG)
    m = jnp.max(seg, axis=1, keepdims=True)
    kstar = jnp.min(jnp.where(seg == m, lane, KPAD), axis=1)
    onehot = (lane == kstar[:, None]).astype(jnp.float32)
    counts = jnp.sum(onehot, axis=0)
    scale = 1.0 / (counts + 1.0)
    sa = a * scale[:, None]
    o_ref[0] = lax.dot_general(
        onehot, sa, (((1,), (0,)), ((), ())), preferred_element_type=jnp.float32
    )


def _aggregate(inp, agg):
    return pl.pallas_call(
        _agg_body,
        grid=(B,),
        in_specs=[
            pl.BlockSpec((1, N, C), lambda b: (b, 0, 0)),
            pl.BlockSpec((1, KPAD, C), lambda b: (b, 0, 0)),
        ],
        out_specs=pl.BlockSpec((1, N, C), lambda b: (b, 0, 0)),
        out_shape=jax.ShapeDtypeStruct((B, N, C), jnp.float32),
    )(inp, agg)


@jax.jit
def kernel(g_feat, input, tau, text_embeddings):
    del tau
    idx = _topk(g_feat[:, 0, :], text_embeddings)
    rows = _sc_gather(text_embeddings, idx.reshape(B * KPAD))
    agg = rows.reshape(B, KPAD, C)
    return _aggregate(input, agg)
